# baseline (speedup 1.0000x reference)
import functools

import jax
import jax.numpy as jnp
from jax import lax
from jax.experimental import pallas as pl
from jax.experimental.pallas import tpu as pltpu
from jax.experimental.pallas import tpu_sc as plsc

BATCH = 16384
SEQ = 500
KMER = 1024
L = 16
NC = 2
NS = 16
NW = NC * NS
CW = BATCH // NW
RBLK = 32
VPR = CW // L
NBLK = SEQ // RBLK
TAIL0 = NBLK * RBLK
TAILR = SEQ - TAIL0


@functools.partial(
    pl.kernel,
    out_type=jax.ShapeDtypeStruct((SEQ, BATCH), jnp.float32),
    mesh=plsc.VectorSubcoreMesh(core_axis_name="c", subcore_axis_name="s"),
    compiler_params=pltpu.CompilerParams(needs_layout_passes=False),
    scratch_types=[
        pltpu.VMEM((KMER,), jnp.float32),
        pltpu.VMEM((528,), jnp.float32),
        pltpu.VMEM((2, RBLK, CW), jnp.int32),
        pltpu.VMEM((2, RBLK, CW), jnp.float32),
        pltpu.VMEM((TAILR, CW), jnp.int32),
        pltpu.VMEM((TAILR, CW), jnp.float32),
        pltpu.SemaphoreType.DMA,
        pltpu.SemaphoreType.DMA,
        pltpu.SemaphoreType.DMA,
        pltpu.SemaphoreType.DMA,
        pltpu.SemaphoreType.DMA,
        pltpu.SemaphoreType.DMA,
    ],
)
def _sc_rates(parents_hbm, ktab_hbm, stab_hbm, out_hbm,
              ket_v, set_v, pin_v, pout_v, tin_v, tout_v,
              isem0, isem1, osem0, osem1, tisem, tosem):
    wid = lax.axis_index("s") * NC + lax.axis_index("c")
    col0 = wid * CW
    isems = (isem0, isem1)
    osems = (osem0, osem1)

    zeros = jnp.zeros((L,), jnp.float32)
    for j in range(SEQ // L * L, 528, L):
        set_v[pl.ds(j, L)] = zeros
    pltpu.sync_copy(ktab_hbm, ket_v)
    pltpu.sync_copy(stab_hbm, set_v.at[pl.ds(0, SEQ)])
    for j in range(KMER // L):
        ket_v[pl.ds(j * L, L)] = jnp.exp(ket_v[pl.ds(j * L, L)])
    for j in range(528 // L):
        set_v[pl.ds(j * L, L)] = jnp.exp(set_v[pl.ds(j * L, L)])

    def start_in(g):
        b = g % 2
        return pltpu.async_copy(
            parents_hbm.at[pl.ds(g * RBLK, RBLK), pl.ds(col0, CW)],
            pin_v.at[b], isems[b])

    def start_out(g):
        b = g % 2
        return pltpu.async_copy(
            pout_v.at[b],
            out_hbm.at[pl.ds(g * RBLK, RBLK), pl.ds(col0, CW)], osems[b])

    def compute_block(pin2, pout2, r0, rc):
        @plsc.parallel_loop(0, rc, unroll=1)
        def row_body(row):
            s = set_v[pl.ds(r0 + row, L)][0]
            for j in range(VPR):
                off = j * L
                idx = pin2[row, pl.ds(off, L)]
                pout2[row, pl.ds(off, L)] = plsc.load_gather(ket_v, [idx]) * s

    tail_in = pltpu.async_copy(
        parents_hbm.at[pl.ds(TAIL0, TAILR), pl.ds(col0, CW)], tin_v, tisem)
    in_d = {0: start_in(0)}
    out_d = {}
    for g in range(NBLK):
        if g + 1 < NBLK:
            in_d[g + 1] = start_in(g + 1)
        in_d[g].wait()
        if g >= 2:
            out_d[g - 2].wait()
        compute_block(pin_v.at[g % 2], pout_v.at[g % 2], g * RBLK, RBLK)
        out_d[g] = start_out(g)
    tail_in.wait()
    compute_block(tin_v, tout_v, TAIL0, TAILR)
    pltpu.async_copy(
        tout_v, out_hbm.at[pl.ds(TAIL0, TAILR), pl.ds(col0, CW)], tosem
    ).wait()
    out_d[NBLK - 2].wait()
    out_d[NBLK - 1].wait()


def kernel(encoded_parents, masks, kmer_emb, site_emb):
    del masks
    out_t = _sc_rates(encoded_parents.T, kmer_emb[:, 0], site_emb[:, 0])
    return out_t.T


# speedup vs baseline: 1.2079x; 1.2079x over previous
import functools

import jax
import jax.numpy as jnp
from jax import lax
from jax.experimental import pallas as pl
from jax.experimental.pallas import tpu as pltpu
from jax.experimental.pallas import tpu_sc as plsc

BATCH = 16384
SEQ = 500
KMER = 1024
L = 16
NC = 2
NS = 16
NW = NC * NS
CW = BATCH // NW
RBLK = 48
VPR = CW // L
NBLK = SEQ // RBLK
TAIL0 = NBLK * RBLK
TAILR = SEQ - TAIL0


@functools.partial(
    pl.kernel,
    out_type=jax.ShapeDtypeStruct((SEQ, BATCH), jnp.float32),
    mesh=plsc.VectorSubcoreMesh(core_axis_name="c", subcore_axis_name="s"),
    compiler_params=pltpu.CompilerParams(needs_layout_passes=False),
    scratch_types=[
        pltpu.VMEM((KMER,), jnp.float32),
        pltpu.VMEM((528,), jnp.float32),
        pltpu.VMEM((2, RBLK, CW), jnp.int32),
        pltpu.VMEM((2, RBLK, CW), jnp.float32),
        pltpu.VMEM((TAILR, CW), jnp.int32),
        pltpu.VMEM((TAILR, CW), jnp.float32),
        pltpu.SemaphoreType.DMA,
        pltpu.SemaphoreType.DMA,
        pltpu.SemaphoreType.DMA,
        pltpu.SemaphoreType.DMA,
        pltpu.SemaphoreType.DMA,
        pltpu.SemaphoreType.DMA,
    ],
)
def _sc_rates(parents_hbm, ktab_hbm, stab_hbm, out_hbm,
              ket_v, set_v, pin_v, pout_v, tin_v, tout_v,
              isem0, isem1, osem0, osem1, tisem, tosem):
    wid = lax.axis_index("s") * NC + lax.axis_index("c")
    col0 = wid * CW
    isems = (isem0, isem1)
    osems = (osem0, osem1)

    zeros = jnp.zeros((L,), jnp.float32)
    for j in range(SEQ // L * L, 528, L):
        set_v[pl.ds(j, L)] = zeros
    pltpu.sync_copy(ktab_hbm, ket_v)
    pltpu.sync_copy(stab_hbm, set_v.at[pl.ds(0, SEQ)])
    for j in range(KMER // L):
        ket_v[pl.ds(j * L, L)] = jnp.exp(ket_v[pl.ds(j * L, L)])
    for j in range(528 // L):
        set_v[pl.ds(j * L, L)] = jnp.exp(set_v[pl.ds(j * L, L)])

    def start_in(g):
        b = g % 2
        return pltpu.async_copy(
            parents_hbm.at[pl.ds(g * RBLK, RBLK), pl.ds(col0, CW)],
            pin_v.at[b], isems[b])

    def start_out(g):
        b = g % 2
        return pltpu.async_copy(
            pout_v.at[b],
            out_hbm.at[pl.ds(g * RBLK, RBLK), pl.ds(col0, CW)], osems[b])

    def compute_block(pin2, pout2, r0, rc):
        def row_body(row, carry):
            s = set_v[pl.ds(r0 + row, L)][0]

            @plsc.parallel_loop(0, VPR, unroll=8)
            def vec_body(j):
                off = j << 4
                idx = pin2[row, pl.ds(off, L)]
                pout2[row, pl.ds(off, L)] = plsc.load_gather(ket_v, [idx]) * s

            return carry

        lax.fori_loop(0, rc, row_body, 0)

    tail_in = pltpu.async_copy(
        parents_hbm.at[pl.ds(TAIL0, TAILR), pl.ds(col0, CW)], tin_v, tisem)
    in_d = {0: start_in(0)}
    out_d = {}
    for g in range(NBLK):
        if g + 1 < NBLK:
            in_d[g + 1] = start_in(g + 1)
        in_d[g].wait()
        if g >= 2:
            out_d[g - 2].wait()
        compute_block(pin_v.at[g % 2], pout_v.at[g % 2], g * RBLK, RBLK)
        out_d[g] = start_out(g)
    tail_in.wait()
    compute_block(tin_v, tout_v, TAIL0, TAILR)
    pltpu.async_copy(
        tout_v, out_hbm.at[pl.ds(TAIL0, TAILR), pl.ds(col0, CW)], tosem
    ).wait()
    out_d[NBLK - 2].wait()
    out_d[NBLK - 1].wait()


def kernel(encoded_parents, masks, kmer_emb, site_emb):
    del masks
    out_t = _sc_rates(encoded_parents.T, kmer_emb[:, 0], site_emb[:, 0])
    return out_t.T
